# baseline (speedup 1.0000x reference)
import functools

import jax
import jax.numpy as jnp
from jax import lax
from jax.experimental import pallas as pl
from jax.experimental.pallas import tpu as pltpu
from jax.experimental.pallas import tpu_sc as plsc

BATCH = 16384
EMB = 32
NC = 2
NS = 16
NW = NC * NS
B_PER_W = BATCH // NW
CHUNK = 128
NCHUNK = B_PER_W // CHUNK


def _gather_body(user_table, product_table, uidx, pidx, u_out, p_out,
                 uidx_v, pidx_v, urows_v, prows_v, sem_u, sem_p):
    wid = lax.axis_index("c") * NS + lax.axis_index("s")
    pltpu.sync_copy(uidx.at[wid], uidx_v)
    pltpu.sync_copy(pidx.at[wid], pidx_v)
    copies = []
    for j in range(NCHUNK):
        copies.append(pltpu.async_copy(user_table.at[uidx_v.at[j]],
                                       urows_v.at[j], sem_u))
    for j in range(NCHUNK):
        copies.append(pltpu.async_copy(product_table.at[pidx_v.at[j]],
                                       prows_v.at[j], sem_p))
    for c in copies:
        c.wait()
    pltpu.sync_copy(urows_v, u_out.at[wid])
    pltpu.sync_copy(prows_v, p_out.at[wid])


def _sc_gather(user_table, product_table, uidx, pidx):
    mesh = plsc.VectorSubcoreMesh(core_axis_name="c", subcore_axis_name="s")
    f = pl.kernel(
        _gather_body,
        out_type=(
            jax.ShapeDtypeStruct((NW, NCHUNK, CHUNK, EMB), jnp.float32),
            jax.ShapeDtypeStruct((NW, NCHUNK, CHUNK, EMB), jnp.float32),
        ),
        mesh=mesh,
        scratch_types=[
            pltpu.VMEM((NCHUNK, CHUNK), jnp.int32),
            pltpu.VMEM((NCHUNK, CHUNK), jnp.int32),
            pltpu.VMEM((NCHUNK, CHUNK, EMB), jnp.float32),
            pltpu.VMEM((NCHUNK, CHUNK, EMB), jnp.float32),
            pltpu.SemaphoreType.DMA,
            pltpu.SemaphoreType.DMA,
        ],
        compiler_params=pltpu.CompilerParams(use_tc_tiling_on_sc=False),
    )
    return f(user_table, product_table, uidx, pidx)


def _mlp_body(u_ref, p_ref, w1u_ref, w1p_ref, b1_ref, w2_ref, b2_ref,
              w3_ref, b3_ref, out_ref):
    h1 = jnp.dot(u_ref[...], w1u_ref[...], preferred_element_type=jnp.float32)
    h1 += jnp.dot(p_ref[...], w1p_ref[...], preferred_element_type=jnp.float32)
    h1 = jnp.maximum(h1 + b1_ref[...], 0.0)
    h2 = jnp.dot(h1, w2_ref[...], preferred_element_type=jnp.float32)
    h2 = jnp.maximum(h2 + b2_ref[...], 0.0)
    out_ref[...] = (jnp.dot(h2, w3_ref[...], preferred_element_type=jnp.float32)
                    + b3_ref[...])


def _tc_mlp(u, p, w1u_t, w1p_t, b1, w2_t, b2, w3_t, b3):
    blk = 2048
    grid = (BATCH // blk,)
    full = lambda shape: pl.BlockSpec(shape, lambda i: (0,) * len(shape))
    return pl.pallas_call(
        _mlp_body,
        grid=grid,
        in_specs=[
            pl.BlockSpec((blk, EMB), lambda i: (i, 0)),
            pl.BlockSpec((blk, EMB), lambda i: (i, 0)),
            full((EMB, 64)),
            full((EMB, 64)),
            full((1, 64)),
            full((64, 32)),
            full((1, 32)),
            full((32, 1)),
            full((1, 1)),
        ],
        out_specs=pl.BlockSpec((blk, 1), lambda i: (i, 0)),
        out_shape=jax.ShapeDtypeStruct((BATCH, 1), jnp.float32),
    )(u, p, w1u_t, w1p_t, b1, w2_t, b2, w3_t, b3)


def kernel(user_tensor, product_tensor, user_table, product_table,
           W1, b1, W2, b2, W3, b3):
    uidx = jnp.reshape(user_tensor.astype(jnp.int32), (NW, NCHUNK, CHUNK))
    pidx = jnp.reshape(product_tensor.astype(jnp.int32), (NW, NCHUNK, CHUNK))
    u_rows, p_rows = _sc_gather(user_table, product_table, uidx, pidx)
    u = jnp.reshape(u_rows, (BATCH, EMB))
    p = jnp.reshape(p_rows, (BATCH, EMB))
    out = _tc_mlp(
        u, p,
        W1[:, :EMB].T, W1[:, EMB:].T, b1[None, :],
        W2.T, b2[None, :], W3.T, b3[None, :],
    )
    return jnp.squeeze(out, axis=-1)


# speedup vs baseline: 1.6375x; 1.6375x over previous
import functools

import jax
import jax.numpy as jnp
from jax import lax
from jax.experimental import pallas as pl
from jax.experimental.pallas import tpu as pltpu
from jax.experimental.pallas import tpu_sc as plsc

BATCH = 16384
EMB = 32
PACK = 4
NC = 2
NS = 16
NW = NC * NS
B_PER_W = BATCH // NW
ROWS_W = B_PER_W // PACK


def _gather_body(user_table, product_table, uidx, pidx, u_out, p_out,
                 uidx_v, pidx_v, urows_v, prows_v, sem_u, sem_p):
    wid = lax.axis_index("c") * NS + lax.axis_index("s")
    base = wid * B_PER_W
    pltpu.sync_copy(uidx.at[pl.ds(base, B_PER_W)], uidx_v)
    pltpu.sync_copy(pidx.at[pl.ds(base, B_PER_W)], pidx_v)

    def chunk_body(c, _):
        cb = pl.multiple_of(c * 16, 16)
        uchunk = uidx_v[pl.ds(cb, 16)]
        pchunk = pidx_v[pl.ds(cb, 16)]
        for k in range(16):
            rb = c * (16 // PACK) + k // PACK
            off = (k % PACK) * EMB
            pltpu.async_copy(user_table.at[uchunk[k]],
                             urows_v.at[rb, pl.ds(off, EMB)], sem_u)
            pltpu.async_copy(product_table.at[pchunk[k]],
                             prows_v.at[rb, pl.ds(off, EMB)], sem_p)
        return ()

    lax.fori_loop(0, B_PER_W // 16, chunk_body, ())
    pltpu.make_async_copy(u_out.at[wid], urows_v, sem_u).wait()
    pltpu.make_async_copy(p_out.at[wid], prows_v, sem_p).wait()
    pltpu.sync_copy(urows_v, u_out.at[wid])
    pltpu.sync_copy(prows_v, p_out.at[wid])


def _sc_gather(user_table, product_table, uidx, pidx):
    mesh = plsc.VectorSubcoreMesh(core_axis_name="c", subcore_axis_name="s")
    f = pl.kernel(
        _gather_body,
        out_type=(
            jax.ShapeDtypeStruct((NW, ROWS_W, PACK * EMB), jnp.float32),
            jax.ShapeDtypeStruct((NW, ROWS_W, PACK * EMB), jnp.float32),
        ),
        mesh=mesh,
        scratch_types=[
            pltpu.VMEM((B_PER_W,), jnp.int32),
            pltpu.VMEM((B_PER_W,), jnp.int32),
            pltpu.VMEM((ROWS_W, PACK * EMB), jnp.float32),
            pltpu.VMEM((ROWS_W, PACK * EMB), jnp.float32),
            pltpu.SemaphoreType.DMA,
            pltpu.SemaphoreType.DMA,
        ],
        compiler_params=pltpu.CompilerParams(use_tc_tiling_on_sc=True),
    )
    return f(user_table, product_table, uidx, pidx)


def _mlp_body(u_ref, p_ref, w1u_ref, w1p_ref, b1_ref, w2_ref, b2_ref,
              w3_ref, b3_ref, out_ref):
    cols = []
    for k in range(PACK):
        uk = u_ref[:, k * EMB:(k + 1) * EMB]
        pk = p_ref[:, k * EMB:(k + 1) * EMB]
        h1 = jnp.dot(uk, w1u_ref[...], preferred_element_type=jnp.float32)
        h1 += jnp.dot(pk, w1p_ref[...], preferred_element_type=jnp.float32)
        h1 = jnp.maximum(h1 + b1_ref[...], 0.0)
        h2 = jnp.dot(h1, w2_ref[...], preferred_element_type=jnp.float32)
        h2 = jnp.maximum(h2 + b2_ref[...], 0.0)
        cols.append(jnp.dot(h2, w3_ref[...],
                            preferred_element_type=jnp.float32))
    out_ref[...] = jnp.concatenate(cols, axis=1) + b3_ref[...]


def _tc_mlp(u, p, w1u_t, w1p_t, b1, w2_t, b2, w3_t, b3):
    n = BATCH // PACK
    blk = 1024
    grid = (n // blk,)
    full = lambda shape: pl.BlockSpec(shape, lambda i: (0,) * len(shape))
    return pl.pallas_call(
        _mlp_body,
        grid=grid,
        in_specs=[
            pl.BlockSpec((blk, PACK * EMB), lambda i: (i, 0)),
            pl.BlockSpec((blk, PACK * EMB), lambda i: (i, 0)),
            full((EMB, 64)),
            full((EMB, 64)),
            full((1, 64)),
            full((64, 32)),
            full((1, 32)),
            full((32, 1)),
            full((1, 1)),
        ],
        out_specs=pl.BlockSpec((blk, PACK), lambda i: (i, 0)),
        out_shape=jax.ShapeDtypeStruct((n, PACK), jnp.float32),
    )(u, p, w1u_t, w1p_t, b1, w2_t, b2, w3_t, b3)


def kernel(user_tensor, product_tensor, user_table, product_table,
           W1, b1, W2, b2, W3, b3):
    uidx = user_tensor.astype(jnp.int32)
    pidx = product_tensor.astype(jnp.int32)
    u_rows, p_rows = _sc_gather(user_table, product_table, uidx, pidx)
    u = jnp.reshape(u_rows, (BATCH // PACK, PACK * EMB))
    p = jnp.reshape(p_rows, (BATCH // PACK, PACK * EMB))
    out = _tc_mlp(
        u, p,
        W1[:, :EMB].T, W1[:, EMB:].T, b1[None, :],
        W2.T, b2[None, :], W3.T, b3[None, :],
    )
    return jnp.reshape(out, (BATCH,))
